# baseline (device time: 13052 ns/iter reference)
import os

import jax
import jax.numpy as jnp
from jax import lax
from jax.experimental import pallas as pl
from jax.experimental.pallas import tpu as pltpu

N_DEV = 4
PROBE = os.environ.get("SCPROBE", "E")


def kernel(x):
    m_per, n = x.shape

    def body(x_ref, out_ref, send_sems, recv_sems):
        me = lax.axis_index("i")
        right = lax.rem(me + 1, N_DEV)
        left = lax.rem(me + N_DEV - 1, N_DEV)

        barrier = pltpu.get_barrier_semaphore()
        for nbr in (left, right):
            pl.semaphore_signal(
                barrier, inc=1,
                device_id=(nbr,), device_id_type=pl.DeviceIdType.MESH,
            )
        pl.semaphore_wait(barrier, 2)

        def sl(origin):
            return out_ref.at[pl.ds(origin * m_per, m_per), :]

        sl(me)[...] = x_ref[:, :].astype(out_ref.dtype)

        if PROBE == "E":
            def make(idx, origin, target):
                return pltpu.make_async_remote_copy(
                    src_ref=sl(origin), dst_ref=sl(origin),
                    send_sem=send_sems.at[idx], recv_sem=recv_sems.at[idx],
                    device_id=(target,), device_id_type=pl.DeviceIdType.MESH,
                )

            s_r = make(0, me, right)
            s_l = make(1, me, left)
            r_l = make(0, left, left)
            r_r = make(1, right, right)
            if os.environ.get("SCPROBE_UNI"):
                s_r.start()
                r_l.wait_recv()
                s_r.wait_send()
            else:
                s_r.start()
                s_l.start()
                r_l.wait_recv()
                r_r.wait_recv()
                s_r.wait_send()
                s_l.wait_send()

    return pl.pallas_call(
        body,
        out_shape=jax.ShapeDtypeStruct((N_DEV * m_per, n), jnp.bfloat16),
        in_specs=[pl.BlockSpec(memory_space=pltpu.VMEM)],
        out_specs=pl.BlockSpec(memory_space=pltpu.VMEM),
        scratch_shapes=[
            pltpu.SemaphoreType.DMA((2,)),
            pltpu.SemaphoreType.DMA((2,)),
        ],
        compiler_params=pltpu.CompilerParams(collective_id=0),
    )(x)


# device time: 9049 ns/iter; 1.4424x vs baseline; 1.4424x over previous
import os

import jax
import jax.numpy as jnp
from jax import lax
from jax.experimental import pallas as pl
from jax.experimental.pallas import tpu as pltpu

N_DEV = 4
PROBE = os.environ.get("SCPROBE", "E")


def kernel(x):
    m_per, n = x.shape

    def body(x_ref, out_ref, send_sems, recv_sems):
        me = lax.axis_index("i")
        right = lax.rem(me + 1, N_DEV)
        left = lax.rem(me + N_DEV - 1, N_DEV)

        barrier = pltpu.get_barrier_semaphore()
        for nbr in (left, right):
            pl.semaphore_signal(
                barrier, inc=1,
                device_id=(nbr,), device_id_type=pl.DeviceIdType.MESH,
            )
        pl.semaphore_wait(barrier, 2)

        rows = m_per // 2 if os.environ.get("SCPROBE_HALF") else m_per

        def sl(origin):
            return out_ref.at[pl.ds(origin * m_per, rows), :]

        sl(me)[...] = x_ref[pl.ds(0, rows), :].astype(out_ref.dtype)

        if PROBE == "E":
            def make(idx, origin, target):
                return pltpu.make_async_remote_copy(
                    src_ref=sl(origin), dst_ref=sl(origin),
                    send_sem=send_sems.at[idx], recv_sem=recv_sems.at[idx],
                    device_id=(target,), device_id_type=pl.DeviceIdType.MESH,
                )

            s_r = make(0, me, right)
            s_l = make(1, me, left)
            r_l = make(0, left, left)
            r_r = make(1, right, right)
            if os.environ.get("SCPROBE_UNI"):
                s_r.start()
                r_l.wait_recv()
                s_r.wait_send()
            else:
                s_r.start()
                s_l.start()
                r_l.wait_recv()
                r_r.wait_recv()
                s_r.wait_send()
                s_l.wait_send()

    return pl.pallas_call(
        body,
        out_shape=jax.ShapeDtypeStruct((N_DEV * m_per, n), jnp.bfloat16),
        in_specs=[pl.BlockSpec(memory_space=pltpu.VMEM)],
        out_specs=pl.BlockSpec(memory_space=pltpu.VMEM),
        scratch_shapes=[
            pltpu.SemaphoreType.DMA((2,)),
            pltpu.SemaphoreType.DMA((2,)),
        ],
        compiler_params=pltpu.CompilerParams(collective_id=0),
    )(x)
